# baseline (speedup 1.0000x reference)
import functools

import jax
import jax.numpy as jnp
from jax import lax
from jax.experimental import pallas as pl
from jax.experimental.pallas import tpu as pltpu
from jax.experimental.pallas import tpu_sc as plsc

N = 50000
DEG = 5
D = 128
DE = 16
H = 8
DIM = 64
E = N * DEG
HEAD = D // H
SCALE = HEAD ** -0.5

_C = 80
_NCHUNK = E // _C
_NW = 32
_NBUF = 8
_CPW = 104
_EP = _NW * _CPW * _C

_TC_BN = 1000


def _sc_gather(table, idx3):
    mesh = plsc.VectorSubcoreMesh(core_axis_name="c", subcore_axis_name="s")

    @functools.partial(
        pl.kernel,
        mesh=mesh,
        out_type=jax.ShapeDtypeStruct((E, D), jnp.float32),
        scratch_types=[
            pltpu.VMEM((_CPW, _C), jnp.int32),
        ]
        + [pltpu.VMEM((_C, D), jnp.float32) for _ in range(_NBUF)]
        + [pltpu.SemaphoreType.DMA, pltpu.SemaphoreType.DMA],
    )
    def k(idx_hbm, table_hbm, out_hbm, idx_v, *rest):
        rows = rest[:_NBUF]
        gsem, wsem = rest[_NBUF], rest[_NBUF + 1]
        wid = lax.axis_index("s") * 2 + lax.axis_index("c")
        pltpu.sync_copy(idx_hbm.at[wid], idx_v)

        def body(t, carry):
            base = wid * _CPW + t * _NBUF
            for b in range(_NBUF):
                @pl.when(base + b < _NCHUNK)
                def _(b=b):
                    pltpu.async_copy(
                        table_hbm.at[idx_v.at[t * _NBUF + b]], rows[b], gsem)
            for b in range(_NBUF):
                @pl.when(base + b < _NCHUNK)
                def _(b=b):
                    pltpu.make_async_copy(
                        table_hbm.at[idx_v.at[t * _NBUF + b]], rows[b],
                        gsem).wait()
                    pltpu.async_copy(
                        rows[b], out_hbm.at[pl.ds((base + b) * _C, _C)], wsem)
            for b in range(_NBUF):
                @pl.when(base + b < _NCHUNK)
                def _(b=b):
                    pltpu.make_async_copy(
                        rows[b], out_hbm.at[pl.ds((base + b) * _C, _C)],
                        wsem).wait()
            return carry

        lax.fori_loop(0, _CPW // _NBUF, body, 0)

    return k(idx3, table)


def _ln(x, g, b):
    m = jnp.mean(x, axis=-1, keepdims=True)
    xc = x - m
    v = jnp.mean(xc * xc, axis=-1, keepdims=True)
    return xc * lax.rsqrt(v + 1e-5) * g + b


def _dot(a, b):
    return jnp.dot(a, b, preferred_element_type=jnp.float32)


def _attn_layer(x, sa, k1s, vps, negs, P):
    (Wq, Wo, bo, Wl1, bl1, Wl2, bl2, g1, be1, g2, be2) = sa
    q1 = _dot(x, Wq[...])
    dots = [
        jnp.where(negs[j], -jnp.inf, _dot(q1 * k1s[j], P))
        for j in range(DEG)
    ]
    m = dots[0]
    for j in range(1, DEG):
        m = jnp.maximum(m, dots[j])
    es = [jnp.exp(d - m) for d in dots]
    tot = es[0]
    for j in range(1, DEG):
        tot = tot + es[j]
    attn = None
    for j in range(DEG):
        contrib = es[j] * vps[j]
        attn = contrib if attn is None else attn + contrib
    attn = attn / tot
    o = _dot(attn, Wo[...]) + bo[...]
    x = x + o
    x = _ln(x, g1[...], be1[...])
    ff = _dot(jnp.maximum(_dot(x, Wl1[...]) + bl1[...], 0.0), Wl2[...]) + bl2[...]
    x = x + ff
    return _ln(x, g2[...], be2[...])


def _tc_body(*refs):
    feat_ref = refs[0]
    gath_ref = refs[1]
    ef_ref = refs[2]
    w = refs[3:-1]
    out_ref = refs[-1]
    Wproj, W1ball, b1, Wv12 = w[0], w[1], w[2], w[3]
    sa1 = w[4:15]
    sa2 = w[15:26]
    W3, b3 = w[26], w[27]

    rP = lax.broadcasted_iota(jnp.int32, (D, D), 0) // HEAD
    cP = lax.broadcasted_iota(jnp.int32, (D, D), 1) // HEAD
    P = jnp.where(rP == cP, SCALE, 0.0).astype(jnp.float32)

    x0 = feat_ref[...]
    v1e = _dot(ef_ref[...], W1ball[...])
    k1s1, k1s2, vps1, vps2, negs = [], [], [], [], []
    for j in range(DEG):
        g = gath_ref[j]
        gp = _dot(g, Wproj[...])
        negs.append(gp[:, 3 * D:3 * D + 1] == 0.0)
        v1 = gp[:, :D] + v1e[:, j * D:(j + 1) * D] + b1[...]
        k1s1.append(gp[:, D:2 * D])
        k1s2.append(gp[:, 2 * D:3 * D])
        vp = _dot(v1, Wv12[...])
        vps1.append(vp[:, :D])
        vps2.append(vp[:, D:2 * D])

    x = _attn_layer(x0, sa1, k1s1, vps1, negs, P)
    x = _attn_layer(x, sa2, k1s2, vps2, negs, P)
    out_ref[...] = jnp.tanh(_dot(x, W3[...]) + b3[...])


def _full_spec(shape):
    nd = len(shape)
    return pl.BlockSpec(shape, lambda i, _n=nd: (0,) * _n)


def _sa_flat(p):
    return [
        p['Wq'], p['Wo'], p['bo'].reshape(1, D),
        p['Wl1'], p['bl1'].reshape(1, 2 * D), p['Wl2'], p['bl2'].reshape(1, D),
        p['g1'].reshape(1, D), p['be1'].reshape(1, D),
        p['g2'].reshape(1, D), p['be2'].reshape(1, D),
    ]


def _tc_forward(feat, gath3, ef2, params):
    W1b = params['W1'][D:]
    W1ball = jnp.zeros((DEG * DE, DEG * D), jnp.float32)
    for j in range(DEG):
        W1ball = W1ball.at[j * DE:(j + 1) * DE, j * D:(j + 1) * D].set(W1b)
    Wproj = jnp.concatenate(
        [params['W1'][:D], params['sa1']['Wk'], params['sa2']['Wk'],
         jnp.ones((D, 1), jnp.float32)], axis=1)
    Wv12 = jnp.concatenate([params['sa1']['Wv'], params['sa2']['Wv']], axis=1)

    weights = (
        [Wproj, W1ball, params['b1'].reshape(1, D), Wv12]
        + _sa_flat(params['sa1'])
        + _sa_flat(params['sa2'])
        + [params['W3'], params['b3'].reshape(1, DIM)]
    )
    bn = _TC_BN
    nb = N // bn
    in_specs = (
        [
            pl.BlockSpec((bn, D), lambda i: (i, 0)),
            pl.BlockSpec((DEG, bn, D), lambda i: (0, i, 0)),
            pl.BlockSpec((bn, DEG * DE), lambda i: (i, 0)),
        ]
        + [_full_spec(x.shape) for x in weights]
    )
    return pl.pallas_call(
        _tc_body,
        grid=(nb,),
        in_specs=in_specs,
        out_specs=pl.BlockSpec((bn, DIM), lambda i: (i, 0)),
        out_shape=jax.ShapeDtypeStruct((N, DIM), jnp.float32),
    )(feat, gath3, ef2, *weights)


def kernel(feat, edge_index, edge_feat, params):
    src = edge_index[0]
    perm = src.reshape(N, DEG).T.reshape(-1)
    perm = jnp.concatenate([perm, jnp.zeros((_EP - E,), jnp.int32)])
    idx3 = perm.reshape(_NW, _CPW, _C)

    gath = _sc_gather(feat, idx3)
    gath3 = gath.reshape(DEG, N, D)
    ef2 = edge_feat.reshape(N, DEG * DE)

    return _tc_forward(feat, gath3, ef2, params)


# speedup vs baseline: 1.0027x; 1.0027x over previous
import functools

import jax
import jax.numpy as jnp
from jax import lax
from jax.experimental import pallas as pl
from jax.experimental.pallas import tpu as pltpu
from jax.experimental.pallas import tpu_sc as plsc

N = 50000
DEG = 5
D = 128
DE = 16
H = 8
DIM = 64
E = N * DEG
HEAD = D // H
SCALE = HEAD ** -0.5

_C = 80
_CN = _C // DEG
_NCHUNK = E // _C
_NW = 32
_NBUF = 4
_CPW = 100
_EP = _NW * _CPW * _C

_TC_BN = 1000


def _sc_gather(table, idx3):
    mesh = plsc.VectorSubcoreMesh(core_axis_name="c", subcore_axis_name="s")

    @functools.partial(
        pl.kernel,
        mesh=mesh,
        out_type=jax.ShapeDtypeStruct((E, D), jnp.float32),
        scratch_types=[
            pltpu.VMEM((_CPW, _C), jnp.int32),
        ]
        + [pltpu.VMEM((_C, D), jnp.float32) for _ in range(_NBUF)]
        + [pltpu.SemaphoreType.DMA, pltpu.SemaphoreType.DMA],
    )
    def k(idx_hbm, table_hbm, out_hbm, idx_v, *rest):
        rows = rest[:_NBUF]
        gsem, wsem = rest[_NBUF], rest[_NBUF + 1]
        wid = lax.axis_index("s") * 2 + lax.axis_index("c")
        pltpu.sync_copy(idx_hbm.at[wid], idx_v)

        def body(t, carry):
            base = wid * _CPW + t * _NBUF
            for b in range(_NBUF):
                @pl.when(base + b < _NCHUNK)
                def _(b=b):
                    pltpu.async_copy(
                        table_hbm.at[idx_v.at[t * _NBUF + b]], rows[b], gsem)
            for b in range(_NBUF):
                @pl.when(base + b < _NCHUNK)
                def _(b=b):
                    pltpu.make_async_copy(
                        table_hbm.at[idx_v.at[t * _NBUF + b]], rows[b],
                        gsem).wait()
                    pltpu.async_copy(
                        rows[b], out_hbm.at[pl.ds((base + b) * _C, _C)], wsem)
            for b in range(_NBUF):
                @pl.when(base + b < _NCHUNK)
                def _(b=b):
                    pltpu.make_async_copy(
                        rows[b], out_hbm.at[pl.ds((base + b) * _C, _C)],
                        wsem).wait()
            return carry

        lax.fori_loop(0, _CPW // _NBUF, body, 0)

    return k(idx3, table)


def _ln(x, g, b):
    m = jnp.mean(x, axis=-1, keepdims=True)
    xc = x - m
    v = jnp.mean(xc * xc, axis=-1, keepdims=True)
    return xc * lax.rsqrt(v + 1e-5) * g + b


def _dot(a, b):
    return jnp.dot(a, b, preferred_element_type=jnp.float32)


def _attn_layer(x, sa, k1s, vps, negs, P):
    (Wq, Wo, bo, Wl1, bl1, Wl2, bl2, g1, be1, g2, be2) = sa
    q1 = _dot(x, Wq[...])
    dots = [
        jnp.where(negs[j], -jnp.inf, _dot(q1 * k1s[j], P))
        for j in range(DEG)
    ]
    m = dots[0]
    for j in range(1, DEG):
        m = jnp.maximum(m, dots[j])
    es = [jnp.exp(d - m) for d in dots]
    tot = es[0]
    for j in range(1, DEG):
        tot = tot + es[j]
    attn = None
    for j in range(DEG):
        contrib = es[j] * vps[j]
        attn = contrib if attn is None else attn + contrib
    attn = attn / tot
    o = _dot(attn, Wo[...]) + bo[...]
    x = x + o
    x = _ln(x, g1[...], be1[...])
    ff = _dot(jnp.maximum(_dot(x, Wl1[...]) + bl1[...], 0.0), Wl2[...]) + bl2[...]
    x = x + ff
    return _ln(x, g2[...], be2[...])


def _tc_body(*refs):
    feat_ref = refs[0]
    gath_ref = refs[1]
    ef_ref = refs[2]
    w = refs[3:-1]
    out_ref = refs[-1]
    Wproj, W1ball, b1, Wv12 = w[0], w[1], w[2], w[3]
    sa1 = w[4:15]
    sa2 = w[15:26]
    W3, b3 = w[26], w[27]

    rP = lax.broadcasted_iota(jnp.int32, (D, D), 0) // HEAD
    cP = lax.broadcasted_iota(jnp.int32, (D, D), 1) // HEAD
    P = jnp.where(rP == cP, SCALE, 0.0).astype(jnp.float32)

    x0 = feat_ref[...]
    v1e = _dot(ef_ref[...], W1ball[...])
    k1s1, k1s2, vps1, vps2, negs = [], [], [], [], []
    for j in range(DEG):
        g = gath_ref[j]
        gp = _dot(g, Wproj[...])
        negs.append(gp[:, 3 * D:3 * D + 1] == 0.0)
        v1 = gp[:, :D] + v1e[:, j * D:(j + 1) * D] + b1[...]
        k1s1.append(gp[:, D:2 * D])
        k1s2.append(gp[:, 2 * D:3 * D])
        vp = _dot(v1, Wv12[...])
        vps1.append(vp[:, :D])
        vps2.append(vp[:, D:2 * D])

    x = _attn_layer(x0, sa1, k1s1, vps1, negs, P)
    x = _attn_layer(x, sa2, k1s2, vps2, negs, P)
    out_ref[...] = jnp.tanh(_dot(x, W3[...]) + b3[...])


def _full_spec(shape):
    nd = len(shape)
    return pl.BlockSpec(shape, lambda i, _n=nd: (0,) * _n)


def _sa_flat(p):
    return [
        p['Wq'], p['Wo'], p['bo'].reshape(1, D),
        p['Wl1'], p['bl1'].reshape(1, 2 * D), p['Wl2'], p['bl2'].reshape(1, D),
        p['g1'].reshape(1, D), p['be1'].reshape(1, D),
        p['g2'].reshape(1, D), p['be2'].reshape(1, D),
    ]


def _tc_forward(feat, gath3, ef2, params):
    W1b = params['W1'][D:]
    W1ball = jnp.zeros((DEG * DE, DEG * D), jnp.float32)
    for j in range(DEG):
        W1ball = W1ball.at[j * DE:(j + 1) * DE, j * D:(j + 1) * D].set(W1b)
    Wproj = jnp.concatenate(
        [params['W1'][:D], params['sa1']['Wk'], params['sa2']['Wk'],
         jnp.ones((D, 1), jnp.float32)], axis=1)
    Wv12 = jnp.concatenate([params['sa1']['Wv'], params['sa2']['Wv']], axis=1)

    weights = (
        [Wproj, W1ball, params['b1'].reshape(1, D), Wv12]
        + _sa_flat(params['sa1'])
        + _sa_flat(params['sa2'])
        + [params['W3'], params['b3'].reshape(1, DIM)]
    )
    bn = _TC_BN
    nb = N // bn
    in_specs = (
        [
            pl.BlockSpec((bn, D), lambda i: (i, 0)),
            pl.BlockSpec((DEG, bn, D), lambda i: (0, i, 0)),
            pl.BlockSpec((bn, DEG * DE), lambda i: (i, 0)),
        ]
        + [_full_spec(x.shape) for x in weights]
    )
    return pl.pallas_call(
        _tc_body,
        grid=(nb,),
        in_specs=in_specs,
        out_specs=pl.BlockSpec((bn, DIM), lambda i: (i, 0)),
        out_shape=jax.ShapeDtypeStruct((N, DIM), jnp.float32),
    )(feat, gath3, ef2, *weights)


def kernel(feat, edge_index, edge_feat, params):
    src = edge_index[0]
    perm = src.reshape(N, DEG).T.reshape(-1)
    perm = jnp.concatenate([perm, jnp.zeros((_EP - E,), jnp.int32)])
    idx3 = perm.reshape(_NW, _CPW, _C)

    gath = _sc_gather(feat, idx3)
    gath3 = gath.reshape(DEG, N, D)
    ef2 = edge_feat.reshape(N, DEG * DE)

    return _tc_forward(feat, gath3, ef2, params)


# speedup vs baseline: 1.0932x; 1.0903x over previous
import functools

import jax
import jax.numpy as jnp
from jax import lax
from jax.experimental import pallas as pl
from jax.experimental.pallas import tpu as pltpu
from jax.experimental.pallas import tpu_sc as plsc

N = 50000
DEG = 5
D = 128
DE = 16
H = 8
DIM = 64
E = N * DEG
HEAD = D // H
SCALE = HEAD ** -0.5

_C = 80
_CN = _C // DEG
_NCHUNK = E // _C
_NW = 32
_NBUF = 4
_CPW = 100
_EP = _NW * _CPW * _C

_TC_BN = 2000


def _sc_gather(table, idx3):
    mesh = plsc.VectorSubcoreMesh(core_axis_name="c", subcore_axis_name="s")

    @functools.partial(
        pl.kernel,
        mesh=mesh,
        out_type=jax.ShapeDtypeStruct((E, D), jnp.float32),
        scratch_types=[
            pltpu.VMEM((_CPW, _C), jnp.int32),
        ]
        + [pltpu.VMEM((_C, D), jnp.float32) for _ in range(_NBUF)]
        + [pltpu.SemaphoreType.DMA, pltpu.SemaphoreType.DMA],
    )
    def k(idx_hbm, table_hbm, out_hbm, idx_v, *rest):
        rows = rest[:_NBUF]
        gsem, wsem = rest[_NBUF], rest[_NBUF + 1]
        wid = lax.axis_index("s") * 2 + lax.axis_index("c")
        pltpu.sync_copy(idx_hbm.at[wid], idx_v)

        def body(t, carry):
            base = wid * _CPW + t * _NBUF
            for b in range(_NBUF):
                @pl.when(base + b < _NCHUNK)
                def _(b=b):
                    pltpu.async_copy(
                        table_hbm.at[idx_v.at[t * _NBUF + b]], rows[b], gsem)
            for b in range(_NBUF):
                @pl.when(base + b < _NCHUNK)
                def _(b=b):
                    pltpu.make_async_copy(
                        table_hbm.at[idx_v.at[t * _NBUF + b]], rows[b],
                        gsem).wait()
                    pltpu.async_copy(
                        rows[b], out_hbm.at[pl.ds((base + b) * _C, _C)], wsem)
            for b in range(_NBUF):
                @pl.when(base + b < _NCHUNK)
                def _(b=b):
                    pltpu.make_async_copy(
                        rows[b], out_hbm.at[pl.ds((base + b) * _C, _C)],
                        wsem).wait()
            return carry

        lax.fori_loop(0, _CPW // _NBUF, body, 0)

    return k(idx3, table)


def _ln(x, g, b):
    m = jnp.mean(x, axis=-1, keepdims=True)
    xc = x - m
    v = jnp.mean(xc * xc, axis=-1, keepdims=True)
    return xc * lax.rsqrt(v + 1e-5) * g + b


def _dot(a, b):
    return jnp.dot(a, b, preferred_element_type=jnp.float32)


def _attn_layer(x, sa, k1s, vps, negs, P):
    (Wq, Wo, bo, Wl1, bl1, Wl2, bl2, g1, be1, g2, be2) = sa
    q1 = _dot(x, Wq[...])
    dots = [
        jnp.where(negs[j], -jnp.inf, _dot(q1 * k1s[j], P))
        for j in range(DEG)
    ]
    m = dots[0]
    for j in range(1, DEG):
        m = jnp.maximum(m, dots[j])
    es = [jnp.exp(d - m) for d in dots]
    tot = es[0]
    for j in range(1, DEG):
        tot = tot + es[j]
    attn = None
    for j in range(DEG):
        contrib = es[j] * vps[j]
        attn = contrib if attn is None else attn + contrib
    attn = attn / tot
    o = _dot(attn, Wo[...]) + bo[...]
    x = x + o
    x = _ln(x, g1[...], be1[...])
    ff = _dot(jnp.maximum(_dot(x, Wl1[...]) + bl1[...], 0.0), Wl2[...]) + bl2[...]
    x = x + ff
    return _ln(x, g2[...], be2[...])


def _tc_body(*refs):
    feat_ref = refs[0]
    gath_ref = refs[1]
    ef_ref = refs[2]
    w = refs[3:-1]
    out_ref = refs[-1]
    Wproj, W1ball, b1, Wv12 = w[0], w[1], w[2], w[3]
    sa1 = w[4:15]
    sa2 = w[15:26]
    W3, b3 = w[26], w[27]

    rP = lax.broadcasted_iota(jnp.int32, (D, D), 0) // HEAD
    cP = lax.broadcasted_iota(jnp.int32, (D, D), 1) // HEAD
    P = jnp.where(rP == cP, SCALE, 0.0).astype(jnp.float32)

    x0 = feat_ref[...]
    v1e = _dot(ef_ref[...], W1ball[...])
    k1s1, k1s2, vps1, vps2, negs = [], [], [], [], []
    for j in range(DEG):
        g = gath_ref[j]
        gp = _dot(g, Wproj[...])
        negs.append(gp[:, 3 * D:3 * D + 1] == 0.0)
        v1 = gp[:, :D] + v1e[:, j * D:(j + 1) * D] + b1[...]
        k1s1.append(gp[:, D:2 * D])
        k1s2.append(gp[:, 2 * D:3 * D])
        vp = _dot(v1, Wv12[...])
        vps1.append(vp[:, :D])
        vps2.append(vp[:, D:2 * D])

    x = _attn_layer(x0, sa1, k1s1, vps1, negs, P)
    x = _attn_layer(x, sa2, k1s2, vps2, negs, P)
    out_ref[...] = jnp.tanh(_dot(x, W3[...]) + b3[...])


def _full_spec(shape):
    nd = len(shape)
    return pl.BlockSpec(shape, lambda i, _n=nd: (0,) * _n)


def _sa_flat(p):
    return [
        p['Wq'], p['Wo'], p['bo'].reshape(1, D),
        p['Wl1'], p['bl1'].reshape(1, 2 * D), p['Wl2'], p['bl2'].reshape(1, D),
        p['g1'].reshape(1, D), p['be1'].reshape(1, D),
        p['g2'].reshape(1, D), p['be2'].reshape(1, D),
    ]


def _tc_forward(feat, gath3, ef2, params):
    W1b = params['W1'][D:]
    W1ball = jnp.zeros((DEG * DE, DEG * D), jnp.float32)
    for j in range(DEG):
        W1ball = W1ball.at[j * DE:(j + 1) * DE, j * D:(j + 1) * D].set(W1b)
    Wproj = jnp.concatenate(
        [params['W1'][:D], params['sa1']['Wk'], params['sa2']['Wk'],
         jnp.ones((D, 1), jnp.float32)], axis=1)
    Wv12 = jnp.concatenate([params['sa1']['Wv'], params['sa2']['Wv']], axis=1)

    weights = (
        [Wproj, W1ball, params['b1'].reshape(1, D), Wv12]
        + _sa_flat(params['sa1'])
        + _sa_flat(params['sa2'])
        + [params['W3'], params['b3'].reshape(1, DIM)]
    )
    bn = _TC_BN
    nb = N // bn
    in_specs = (
        [
            pl.BlockSpec((bn, D), lambda i: (i, 0)),
            pl.BlockSpec((DEG, bn, D), lambda i: (0, i, 0)),
            pl.BlockSpec((bn, DEG * DE), lambda i: (i, 0)),
        ]
        + [_full_spec(x.shape) for x in weights]
    )
    return pl.pallas_call(
        _tc_body,
        grid=(nb,),
        in_specs=in_specs,
        out_specs=pl.BlockSpec((bn, DIM), lambda i: (i, 0)),
        out_shape=jax.ShapeDtypeStruct((N, DIM), jnp.float32),
    )(feat, gath3, ef2, *weights)


def kernel(feat, edge_index, edge_feat, params):
    src = edge_index[0]
    perm = src.reshape(N, DEG).T.reshape(-1)
    perm = jnp.concatenate([perm, jnp.zeros((_EP - E,), jnp.int32)])
    idx3 = perm.reshape(_NW, _CPW, _C)

    gath = _sc_gather(feat, idx3)
    gath3 = gath.reshape(DEG, N, D)
    ef2 = edge_feat.reshape(N, DEG * DE)

    return _tc_forward(feat, gath3, ef2, params)
